# baseline (speedup 1.0000x reference)
import jax
import jax.numpy as jnp
from jax import lax
from jax.experimental import pallas as pl

_K = 64
_BLOCK_B = 8


def _body(x_ref, o_ref):
    xv = x_ref[...]
    i = lax.bitcast_convert_type(xv, jnp.int32)
    key = i ^ (lax.shift_right_arithmetic(i, 31) & jnp.int32(0x7FFFFFFF))

    sign = jnp.int32(-2147483648)

    def cond(state):
        t, _, cur = state
        return jnp.logical_and(t < 11, jnp.any(cur != _K))

    def count(cand):
        return jnp.sum((key >= (cand ^ sign)).astype(jnp.int32),
                       axis=1, keepdims=True)

    def step(state):
        t, uprefix, cur = state
        b2 = jnp.int32(1) << jnp.maximum(jnp.int32(31) - 3 * t, 0)
        b1 = jnp.int32(1) << jnp.maximum(jnp.int32(30) - 3 * t, 0)
        b0 = jnp.int32(1) << jnp.maximum(jnp.int32(29) - 3 * t, 0)
        cands = [uprefix | b2 | b1 | b0, uprefix | b2 | b1, uprefix | b2 | b0,
                 uprefix | b2, uprefix | b1 | b0, uprefix | b1, uprefix | b0]
        cnts = [count(c) for c in cands]
        newp, newc, taken = uprefix, cur, jnp.zeros_like(cur, dtype=bool)
        for c, n in zip(cands, cnts):
            take = jnp.logical_and(~taken, n >= _K)
            newp = jnp.where(take, c, newp)
            newc = jnp.where(take, n, newc)
            taken = jnp.logical_or(taken, take)
        return (t + 1, newp, newc)

    BB = xv.shape[0]
    init = (jnp.int32(0),
            jnp.zeros((BB, 1), jnp.int32),
            jnp.full((BB, 1), jnp.int32(xv.shape[1])))
    _, uthresh, _ = lax.while_loop(cond, step, init)
    thresh = uthresh ^ sign

    o_ref[...] = jnp.where(key >= thresh, jnp.maximum(xv, 0.0), 0.0)


def kernel(x):
    B, N = x.shape
    grid = (B // _BLOCK_B,)
    return pl.pallas_call(
        _body,
        grid=grid,
        in_specs=[pl.BlockSpec((_BLOCK_B, N), lambda b: (b, 0))],
        out_specs=pl.BlockSpec((_BLOCK_B, N), lambda b: (b, 0)),
        out_shape=jax.ShapeDtypeStruct((B, N), x.dtype),
    )(x)
